# Initial kernel scaffold; baseline (speedup 1.0000x reference)
import jax
import jax.numpy as jnp
from jax.experimental import pallas as pl


def kernel(in_tensor, table):
    raise NotImplementedError("write your pallas kernel here")


# speedup vs baseline: 20.1937x; 20.1937x over previous
import functools

import jax
import jax.numpy as jnp
from jax import lax
from jax.experimental import pallas as pl
from jax.experimental.pallas import tpu as pltpu
from jax.experimental.pallas import tpu_sc as plsc

_IN_DIM = 3
_OUT_DIM = 32
_BIN_NUM = 65536
_N_POINTS = 262144

_NC = 2
_NS = 16
_NW = _NC * _NS
_PPW = _N_POINTS // _NW
_CHUNK = 512
_NCHUNK = _PPW // _CHUNK
_VPA = _CHUNK // 16
_BURSTS = (_IN_DIM * _CHUNK) // 128
_BPA = _CHUNK // 128


def _body(x0_hbm, x1_hbm, x2_hbm, table_hbm, out_hbm,
          x0_v, x1_v, x2_v, idx_v, r0, r1, r2, o_v, sem):
    wid = lax.axis_index("s") * _NC + lax.axis_index("c")
    wbase = wid * _PPW

    xs = (x0_v, x1_v, x2_v)
    for a, xh in enumerate((x0_hbm, x1_hbm, x2_hbm)):
        pltpu.sync_copy(xh.at[pl.ds(wbase, _PPW)], xs[a])

    rows = (r0, r1, r2)

    def chunk_body(ci, carry):
        cbase = ci * _CHUNK

        for a in range(_IN_DIM):
            for v in range(_VPA):
                xv = xs[a][pl.ds(cbase + v * 16, 16)]
                idf = (xv + 1.0) * (0.5 * (_BIN_NUM - 1))
                ii = idf.astype(jnp.int32)
                ii = jnp.maximum(jnp.minimum(ii, _BIN_NUM - 1), 0)
                ii = ii + a * _BIN_NUM
                flat = a * _CHUNK + v * 16
                idx_v[flat // 128, pl.ds(flat % 128, 16)] = ii

        cps = []
        for a in range(_IN_DIM):
            for b in range(_BPA):
                cps.append(
                    pltpu.async_copy(
                        table_hbm.at[idx_v.at[a * _BPA + b]],
                        rows[a].at[pl.ds(b * 128, 128)],
                        sem,
                    )
                )
        for cp in cps:
            cp.wait()

        def mean_body(p, c2):
            for u in range(4):
                for h in range(2):
                    s = pl.ds(h * 16, 16)
                    q = p * 4 + u
                    acc = r0[q, s] + r1[q, s] + r2[q, s]
                    o_v[q, s] = acc * (1.0 / 3.0)
            return c2

        lax.fori_loop(0, _CHUNK // 4, mean_body, 0, unroll=False)

        pltpu.sync_copy(o_v, out_hbm.at[pl.ds(wbase + cbase, _CHUNK)])
        return carry

    lax.fori_loop(0, _NCHUNK, chunk_body, 0, unroll=False)


@jax.jit
def _run(x0, x1, x2, table):
    mesh = plsc.VectorSubcoreMesh(core_axis_name="c", subcore_axis_name="s")
    f = pl.kernel(
        _body,
        out_type=jax.ShapeDtypeStruct((_N_POINTS, _OUT_DIM), jnp.float32),
        mesh=mesh,
        scratch_types=[
            pltpu.VMEM((_PPW,), jnp.float32),
            pltpu.VMEM((_PPW,), jnp.float32),
            pltpu.VMEM((_PPW,), jnp.float32),
            pltpu.VMEM((_BURSTS, 128), jnp.int32),
            pltpu.VMEM((_CHUNK, _OUT_DIM), jnp.float32),
            pltpu.VMEM((_CHUNK, _OUT_DIM), jnp.float32),
            pltpu.VMEM((_CHUNK, _OUT_DIM), jnp.float32),
            pltpu.VMEM((_CHUNK, _OUT_DIM), jnp.float32),
            pltpu.SemaphoreType.DMA,
        ],
        compiler_params=pltpu.CompilerParams(use_tc_tiling_on_sc=False),
    )
    return f(x0, x1, x2, table)


def kernel(in_tensor, table):
    x_t = in_tensor.T
    return _run(x_t[0], x_t[1], x_t[2], table)
